# Initial kernel scaffold; baseline (speedup 1.0000x reference)
import jax
import jax.numpy as jnp
from jax.experimental import pallas as pl


def kernel(x, numericals, Wh, bh, WQ, bQ, WK, bK, WV, bV, WO, bO, ln1_g, ln1_b, ln2_g, ln2_b, W1, b1, W2, b2, curr_emb, pay_emb, mlpW, mlpb, mlpWo, mlpbo, edge_index, payment_currency, receiving_currency, payment_format):
    raise NotImplementedError("write your pallas kernel here")


# speedup vs baseline: 11.8012x; 11.8012x over previous
import functools

import jax
import jax.numpy as jnp
from jax import lax
from jax.experimental import pallas as pl
from jax.experimental.pallas import tpu as pltpu
from jax.experimental.pallas import tpu_sc as plsc

N = 10000
E = 160000
D = 128
H = 8
DH = 16
L = 4
NP = 10240
EP = 163840
NW = 32
EW = EP // NW
CA = 64
NCH = EW // CA
NCH2 = (EP // 16) // CA
RB = 1024
TB = 640
MP = 288


def _inproj_body(x_ref, w_ref, b_ref, o_ref):
    o_ref[...] = jnp.dot(x_ref[...], w_ref[...],
                         preferred_element_type=jnp.float32) + b_ref[...]


def _qkv_body(h_ref, g_ref, b_ref, wq_ref, bq_ref, wk_ref, bk_ref,
              wv_ref, bv_ref, hsel_ref, q_ref, k_ref, v_ref,
              qn_ref, kn_ref):
    h = h_ref[...]
    mu = jnp.mean(h, axis=1, keepdims=True)
    var = jnp.mean((h - mu) ** 2, axis=1, keepdims=True)
    hn = (h - mu) / jnp.sqrt(var + 1e-5) * g_ref[...] + b_ref[...]
    q = jnp.dot(hn, wq_ref[...], preferred_element_type=jnp.float32) + bq_ref[...]
    k = jnp.dot(hn, wk_ref[...], preferred_element_type=jnp.float32) + bk_ref[...]
    v = jnp.dot(hn, wv_ref[...], preferred_element_type=jnp.float32) + bv_ref[...]
    q_ref[...] = q
    k_ref[...] = k
    v_ref[...] = v
    hsel = hsel_ref[...]
    qn_ref[...] = jnp.dot(q * q, hsel, preferred_element_type=jnp.float32)
    kn_ref[...] = jnp.dot(k * k, hsel, preferred_element_type=jnp.float32)


def _post_body(h_ref, acc_ref, den_ref, r8_ref, wo_ref, bo_ref,
               g2_ref, b2_ref, w1_ref, b1_ref, w2_ref, b2m_ref, o_ref):
    acc = acc_ref[...]
    den = den_ref[...]
    denf = jnp.dot(den, r8_ref[...], preferred_element_type=jnp.float32)
    safe = jnp.where(denf > 0.0, denf, 1.0)
    hmha = acc / safe
    u = h_ref[...] + jnp.dot(hmha, wo_ref[...],
                             preferred_element_type=jnp.float32) + bo_ref[...]
    mu = jnp.mean(u, axis=1, keepdims=True)
    var = jnp.mean((u - mu) ** 2, axis=1, keepdims=True)
    n = (u - mu) / jnp.sqrt(var + 1e-5) * g2_ref[...] + b2_ref[...]
    y = jnp.maximum(jnp.dot(n, w1_ref[...],
                            preferred_element_type=jnp.float32) + b1_ref[...], 0.0)
    y = jnp.dot(y, w2_ref[...], preferred_element_type=jnp.float32) + b2m_ref[...]
    o_ref[...] = u + y


def _mlp_body(hs_ref, hd_ref, pc_ref, rc_ref, pf_ref, num_ref,
              w0a_ref, w0b_ref, wpc_ref, wrc_ref, wpf_ref, wnum_ref, b0_ref,
              w1_ref, b1_ref, w2_ref, b2_ref, wo_ref, bo_ref, o_ref):
    f32 = jnp.float32
    it16 = lax.broadcasted_iota(jnp.int32, (TB, 16), 1)
    it8 = lax.broadcasted_iota(jnp.int32, (TB, 8), 1)
    pcoh = (it16 == pc_ref[...]).astype(f32)
    rcoh = (it16 == rc_ref[...]).astype(f32)
    pfoh = (it8 == pf_ref[...]).astype(f32)
    y = (jnp.dot(hs_ref[...], w0a_ref[...], preferred_element_type=f32)
         + jnp.dot(hd_ref[...], w0b_ref[...], preferred_element_type=f32)
         + jnp.dot(pcoh, wpc_ref[...], preferred_element_type=f32)
         + jnp.dot(rcoh, wrc_ref[...], preferred_element_type=f32)
         + jnp.dot(pfoh, wpf_ref[...], preferred_element_type=f32)
         + jnp.dot(num_ref[...], wnum_ref[...], preferred_element_type=f32)
         + b0_ref[...])
    y = jnp.maximum(y, 0.0)
    y = jnp.maximum(jnp.dot(y, w1_ref[...], preferred_element_type=f32)
                    + b1_ref[...], 0.0)
    y = jnp.maximum(jnp.dot(y, w2_ref[...], preferred_element_type=f32)
                    + b2_ref[...], 0.0)
    o_ref[...] = jnp.dot(y, wo_ref[...], preferred_element_type=f32) + bo_ref[...]


def _node_spec(shape):
    return pl.BlockSpec(shape, lambda i: (0,) * len(shape))


_inproj = pl.pallas_call(
    _inproj_body,
    grid=(NP // RB,),
    in_specs=[pl.BlockSpec((RB, D), lambda i: (i, 0)),
              _node_spec((D, D)), _node_spec((1, D))],
    out_specs=pl.BlockSpec((RB, D), lambda i: (i, 0)),
    out_shape=jax.ShapeDtypeStruct((NP, D), jnp.float32),
)

_qkv = pl.pallas_call(
    _qkv_body,
    grid=(NP // RB,),
    in_specs=[pl.BlockSpec((RB, D), lambda i: (i, 0)),
              _node_spec((1, D)), _node_spec((1, D)),
              _node_spec((D, D)), _node_spec((1, D)),
              _node_spec((D, D)), _node_spec((1, D)),
              _node_spec((D, D)), _node_spec((1, D)),
              _node_spec((D, H))],
    out_specs=[pl.BlockSpec((RB, D), lambda i: (i, 0)),
               pl.BlockSpec((RB, D), lambda i: (i, 0)),
               pl.BlockSpec((RB, D), lambda i: (i, 0)),
               pl.BlockSpec((RB, H), lambda i: (i, 0)),
               pl.BlockSpec((RB, H), lambda i: (i, 0))],
    out_shape=[jax.ShapeDtypeStruct((NP, D), jnp.float32),
               jax.ShapeDtypeStruct((NP, D), jnp.float32),
               jax.ShapeDtypeStruct((NP, D), jnp.float32),
               jax.ShapeDtypeStruct((NP, H), jnp.float32),
               jax.ShapeDtypeStruct((NP, H), jnp.float32)],
)

_post = pl.pallas_call(
    _post_body,
    grid=(NP // RB,),
    in_specs=[pl.BlockSpec((RB, D), lambda i: (i, 0)),
              pl.BlockSpec((RB, D), lambda i: (i, 0)),
              pl.BlockSpec((RB, H), lambda i: (i, 0)),
              _node_spec((H, D)),
              _node_spec((D, D)), _node_spec((1, D)),
              _node_spec((1, D)), _node_spec((1, D)),
              _node_spec((D, D)), _node_spec((1, D)),
              _node_spec((D, D)), _node_spec((1, D))],
    out_specs=pl.BlockSpec((RB, D), lambda i: (i, 0)),
    out_shape=jax.ShapeDtypeStruct((NP, D), jnp.float32),
)

_mlp = pl.pallas_call(
    _mlp_body,
    grid=(EP // TB,),
    in_specs=[pl.BlockSpec((TB, D), lambda i: (i, 0)),
              pl.BlockSpec((TB, D), lambda i: (i, 0)),
              pl.BlockSpec((TB, 1), lambda i: (i, 0)),
              pl.BlockSpec((TB, 1), lambda i: (i, 0)),
              pl.BlockSpec((TB, 1), lambda i: (i, 0)),
              pl.BlockSpec((TB, 8), lambda i: (i, 0)),
              _node_spec((D, MP)), _node_spec((D, MP)),
              _node_spec((16, MP)), _node_spec((16, MP)),
              _node_spec((8, MP)), _node_spec((8, MP)), _node_spec((1, MP)),
              _node_spec((MP, MP)), _node_spec((1, MP)),
              _node_spec((MP, MP)), _node_spec((1, MP)),
              _node_spec((MP, 8)), _node_spec((1, 8))],
    out_specs=pl.BlockSpec((TB, 8), lambda i: (i, 0)),
    out_shape=jax.ShapeDtypeStruct((EP, 8), jnp.float32),
)


def _attn_sc_body(q2_hbm, k2_hbm, v2_hbm, src_hbm, dst_hbm, m_hbm,
                  z80_hbm, accden_out,
                  idxs_v, idxd_v, qb, kb, vb, wb, mb,
                  sh, sem1, sem2, sem3):
    c = lax.axis_index("c")
    s = lax.axis_index("s")
    rows = NP // 16
    pltpu.sync_copy(z80_hbm.at[pl.ds(s * rows, rows)],
                    sh.at[pl.ds(s * rows, rows)])
    pltpu.sync_copy(m_hbm, mb)
    plsc.subcore_barrier()

    lanes = lax.iota(jnp.int32, 16)
    mvec = mb[...]

    def lgather(vec, idx):
        return lax.gather(
            vec, idx.reshape(16, 1),
            lax.GatherDimensionNumbers(offset_dims=(),
                                       collapsed_slice_dims=(0,),
                                       start_index_map=(0,)),
            (1,), mode=lax.GatherScatterMode.PROMISE_IN_BOUNDS)

    mh = [lgather(mvec, jnp.full((16,), 4 * c + h, jnp.int32))
          for h in range(H // 2)]
    perms = [lanes ^ (1 << b) for b in range(4)]
    inv = 0.25

    def allsum(p):
        for pm in perms:
            p = p + lgather(p, pm)
        return p

    base0 = s * (EP // 16)

    def chunk_body(ch, carry):
        base = base0 + ch * CA
        pltpu.sync_copy(src_hbm.at[pl.ds(base, CA)], idxs_v)
        pltpu.sync_copy(dst_hbm.at[pl.ds(base, CA)], idxd_v)
        cp1 = pltpu.async_copy(q2_hbm.at[c].at[idxd_v], qb, sem1)
        cp2 = pltpu.async_copy(k2_hbm.at[c].at[idxs_v], kb, sem2)
        cp3 = pltpu.async_copy(v2_hbm.at[c].at[idxs_v], vb, sem3)
        cp1.wait()
        cp2.wait()
        cp3.wait()

        def edge_body(e, _):
            ev = jnp.zeros((16,), jnp.float32)
            for h in range(H // 2):
                sl = pl.ds(h * DH, DH)
                p = qb[e, sl] * kb[e, sl]
                exv = jnp.exp(allsum(p) * inv - mh[h])
                wb[e, sl] = exv * vb[e, sl]
                ev = jnp.where(lanes == h, exv, ev)
            wb[e, pl.ds(64, 16)] = ev
            return 0

        lax.fori_loop(0, CA, edge_body, 0)
        pltpu.sync_copy(wb, sh.at[idxd_v], add=True)
        return carry

    lax.fori_loop(0, NCH2, chunk_body, 0)
    plsc.subcore_barrier()
    pltpu.sync_copy(sh.at[pl.ds(s * rows, rows)],
                    accden_out.at[c, pl.ds(s * rows, rows)])


@functools.cache
def _attn_kernel():
  return pl.kernel(
    _attn_sc_body,
    out_type=jax.ShapeDtypeStruct((2, NP, 80), jnp.float32),
    mesh=plsc.VectorSubcoreMesh(core_axis_name="c", subcore_axis_name="s"),
    compiler_params=pltpu.CompilerParams(use_tc_tiling_on_sc=False),
    scratch_types=[
        pltpu.VMEM((CA,), jnp.int32),
        pltpu.VMEM((CA,), jnp.int32),
        pltpu.VMEM((CA, 64), jnp.float32),
        pltpu.VMEM((CA, 64), jnp.float32),
        pltpu.VMEM((CA, 64), jnp.float32),
        pltpu.VMEM((CA, 80), jnp.float32),
        pltpu.VMEM((16,), jnp.float32),
        pltpu.VMEM_SHARED((NP, 80), jnp.float32),
        pltpu.SemaphoreType.DMA,
        pltpu.SemaphoreType.DMA,
        pltpu.SemaphoreType.DMA,
    ],
  )


def _gather_sc_body(h_hbm, src_hbm, dst_hbm, hs_out, hd_out, idx_v, buf, sem):
    c = lax.axis_index("c")
    s = lax.axis_index("s")
    wid = c * 16 + s
    base0 = wid * EW

    def chunk_body(ch, carry):
        base = base0 + ch * CA
        pltpu.sync_copy(src_hbm.at[pl.ds(base, CA)], idx_v)
        pltpu.async_copy(h_hbm.at[idx_v], buf, sem).wait()
        pltpu.sync_copy(buf, hs_out.at[pl.ds(base, CA)])
        pltpu.sync_copy(dst_hbm.at[pl.ds(base, CA)], idx_v)
        pltpu.async_copy(h_hbm.at[idx_v], buf, sem).wait()
        pltpu.sync_copy(buf, hd_out.at[pl.ds(base, CA)])
        return carry

    lax.fori_loop(0, NCH, chunk_body, 0)


@functools.cache
def _gather_kernel():
  return pl.kernel(
    _gather_sc_body,
    out_type=[jax.ShapeDtypeStruct((EP, D), jnp.float32),
              jax.ShapeDtypeStruct((EP, D), jnp.float32)],
    mesh=plsc.VectorSubcoreMesh(core_axis_name="c", subcore_axis_name="s"),
    scratch_types=[
        pltpu.VMEM((CA,), jnp.int32),
        pltpu.VMEM((CA, D), jnp.float32),
        pltpu.SemaphoreType.DMA,
    ],
  )


def kernel(x, numericals, Wh, bh, WQ, bQ, WK, bK, WV, bV, WO, bO,
           ln1_g, ln1_b, ln2_g, ln2_b, W1, b1, W2, b2, curr_emb, pay_emb,
           mlpW, mlpb, mlpWo, mlpbo, edge_index, payment_currency,
           receiving_currency, payment_format):
    f32 = jnp.float32
    xp = jnp.zeros((NP, D), f32).at[:N].set(x)
    src = edge_index[0]
    dst = edge_index[1]
    pad_e = EP - E
    srcp = jnp.concatenate([src, jnp.zeros((pad_e,), jnp.int32)])
    dstp = jnp.concatenate(
        [dst, N + (jnp.arange(pad_e, dtype=jnp.int32) % 16)])

    hsel = jnp.kron(jnp.eye(H, dtype=f32), jnp.ones((DH, 1), f32))
    r8 = jnp.kron(jnp.eye(H, dtype=f32), jnp.ones((1, DH), f32))
    z80 = jnp.zeros((NP, 80), f32)

    h = _inproj(xp, Wh, bh.reshape(1, D))

    for l in range(L):
        q, k, v, qn, kn = _qkv(h, ln1_g[l].reshape(1, D), ln1_b[l].reshape(1, D),
                               WQ[l], bQ[l].reshape(1, D),
                               WK[l], bK[l].reshape(1, D),
                               WV[l], bV[l].reshape(1, D), hsel)
        m8 = jnp.sqrt(jnp.max(qn, axis=0)) * jnp.sqrt(jnp.max(kn, axis=0)) * 0.25
        mvec = jnp.concatenate([m8, jnp.zeros((8,), f32)])
        q2 = jnp.stack([q[:, :64], q[:, 64:]])
        k2 = jnp.stack([k[:, :64], k[:, 64:]])
        v2 = jnp.stack([v[:, :64], v[:, 64:]])
        accden = _attn_kernel()(q2, k2, v2, srcp, dstp, mvec, z80)
        acc = jnp.concatenate([accden[0, :, :64], accden[1, :, :64]], axis=1)
        den = jnp.concatenate([accden[0, :, 64:68], accden[1, :, 64:68]], axis=1)
        h = _post(h, acc, den, r8,
                  WO[l], bO[l].reshape(1, D),
                  ln2_g[l].reshape(1, D), ln2_b[l].reshape(1, D),
                  W1[l], b1[l].reshape(1, D),
                  W2[l], b2[l].reshape(1, D))

    hs, hd = _gather_kernel()(h, srcp, dstp)

    w0 = mlpW[0]
    w0a = jnp.zeros((D, MP), f32).at[:, :285].set(w0[:D])
    w0b = jnp.zeros((D, MP), f32).at[:, :285].set(w0[D:2 * D])
    wpc = jnp.zeros((16, MP), f32).at[:15, :285].set(curr_emb @ w0[256:264])
    wrc = jnp.zeros((16, MP), f32).at[:15, :285].set(curr_emb @ w0[264:272])
    wpf = jnp.zeros((8, MP), f32).at[:7, :285].set(pay_emb @ w0[272:280])
    wnum = jnp.zeros((8, MP), f32).at[:5, :285].set(w0[280:285])
    b0 = jnp.zeros((1, MP), f32).at[0, :285].set(mlpb[0])
    w1m = jnp.zeros((MP, MP), f32).at[:285, :285].set(mlpW[1])
    b1m = jnp.zeros((1, MP), f32).at[0, :285].set(mlpb[1])
    w2m = jnp.zeros((MP, MP), f32).at[:285, :285].set(mlpW[2])
    b2m = jnp.zeros((1, MP), f32).at[0, :285].set(mlpb[2])
    wom = jnp.zeros((MP, 8), f32).at[:285, :2].set(mlpWo)
    bom = jnp.zeros((1, 8), f32).at[0, :2].set(mlpbo)

    pcp = jnp.concatenate([payment_currency,
                           jnp.zeros((pad_e,), jnp.int32)]).reshape(EP, 1)
    rcp = jnp.concatenate([receiving_currency,
                           jnp.zeros((pad_e,), jnp.int32)]).reshape(EP, 1)
    pfp = jnp.concatenate([payment_format,
                           jnp.zeros((pad_e,), jnp.int32)]).reshape(EP, 1)
    nump = jnp.zeros((EP, 8), f32).at[:E, :5].set(numericals)

    y = _mlp(hs, hd, pcp, rcp, pfp, nump,
             w0a, w0b, wpc, wrc, wpf, wnum, b0,
             w1m, b1m, w2m, b2m, wom, bom)
    return y[:E, :2]


# speedup vs baseline: 13.4924x; 1.1433x over previous
import functools

import jax
import jax.numpy as jnp
from jax import lax
from jax.experimental import pallas as pl
from jax.experimental.pallas import tpu as pltpu
from jax.experimental.pallas import tpu_sc as plsc

N = 10000
E = 160000
D = 128
H = 8
DH = 16
L = 4
NP = 10240
EP = 163840
NW = 32
EW = EP // NW
CA = 40
NCH = EW // CA
NCH2 = (EP // 16) // CA
RB = 1024
TB = 640
MP = 288


def _inproj_body(x_ref, w_ref, b_ref, o_ref):
    o_ref[...] = jnp.dot(x_ref[...], w_ref[...],
                         preferred_element_type=jnp.float32) + b_ref[...]


def _qkv_body(h_ref, g_ref, b_ref, wq_ref, bq_ref, wk_ref, bk_ref,
              wv_ref, bv_ref, hsel_ref, q_ref, k_ref, v_ref,
              qn_ref, kn_ref):
    h = h_ref[...]
    mu = jnp.mean(h, axis=1, keepdims=True)
    var = jnp.mean((h - mu) ** 2, axis=1, keepdims=True)
    hn = (h - mu) / jnp.sqrt(var + 1e-5) * g_ref[...] + b_ref[...]
    q = jnp.dot(hn, wq_ref[...], preferred_element_type=jnp.float32) + bq_ref[...]
    k = jnp.dot(hn, wk_ref[...], preferred_element_type=jnp.float32) + bk_ref[...]
    v = jnp.dot(hn, wv_ref[...], preferred_element_type=jnp.float32) + bv_ref[...]
    q_ref[...] = q
    k_ref[...] = k
    v_ref[...] = v
    hsel = hsel_ref[...]
    qn_ref[...] = jnp.dot(q * q, hsel, preferred_element_type=jnp.float32)
    kn_ref[...] = jnp.dot(k * k, hsel, preferred_element_type=jnp.float32)


def _post_body(h_ref, acc_ref, den_ref, r8_ref, wo_ref, bo_ref,
               g2_ref, b2_ref, w1_ref, b1_ref, w2_ref, b2m_ref, o_ref):
    acc = acc_ref[...]
    den = den_ref[...]
    denf = jnp.dot(den, r8_ref[...], preferred_element_type=jnp.float32)
    safe = jnp.where(denf > 0.0, denf, 1.0)
    hmha = acc / safe
    u = h_ref[...] + jnp.dot(hmha, wo_ref[...],
                             preferred_element_type=jnp.float32) + bo_ref[...]
    mu = jnp.mean(u, axis=1, keepdims=True)
    var = jnp.mean((u - mu) ** 2, axis=1, keepdims=True)
    n = (u - mu) / jnp.sqrt(var + 1e-5) * g2_ref[...] + b2_ref[...]
    y = jnp.maximum(jnp.dot(n, w1_ref[...],
                            preferred_element_type=jnp.float32) + b1_ref[...], 0.0)
    y = jnp.dot(y, w2_ref[...], preferred_element_type=jnp.float32) + b2m_ref[...]
    o_ref[...] = u + y


def _mlp_body(hs_ref, hd_ref, pc_ref, rc_ref, pf_ref, num_ref,
              w0a_ref, w0b_ref, wpc_ref, wrc_ref, wpf_ref, wnum_ref, b0_ref,
              w1_ref, b1_ref, w2_ref, b2_ref, wo_ref, bo_ref, o_ref):
    f32 = jnp.float32
    it16 = lax.broadcasted_iota(jnp.int32, (TB, 16), 1)
    it8 = lax.broadcasted_iota(jnp.int32, (TB, 8), 1)
    pcoh = (it16 == pc_ref[...]).astype(f32)
    rcoh = (it16 == rc_ref[...]).astype(f32)
    pfoh = (it8 == pf_ref[...]).astype(f32)
    y = (jnp.dot(hs_ref[...], w0a_ref[...], preferred_element_type=f32)
         + jnp.dot(hd_ref[...], w0b_ref[...], preferred_element_type=f32)
         + jnp.dot(pcoh, wpc_ref[...], preferred_element_type=f32)
         + jnp.dot(rcoh, wrc_ref[...], preferred_element_type=f32)
         + jnp.dot(pfoh, wpf_ref[...], preferred_element_type=f32)
         + jnp.dot(num_ref[...], wnum_ref[...], preferred_element_type=f32)
         + b0_ref[...])
    y = jnp.maximum(y, 0.0)
    y = jnp.maximum(jnp.dot(y, w1_ref[...], preferred_element_type=f32)
                    + b1_ref[...], 0.0)
    y = jnp.maximum(jnp.dot(y, w2_ref[...], preferred_element_type=f32)
                    + b2_ref[...], 0.0)
    o_ref[...] = jnp.dot(y, wo_ref[...], preferred_element_type=f32) + bo_ref[...]


def _node_spec(shape):
    return pl.BlockSpec(shape, lambda i: (0,) * len(shape))


_inproj = pl.pallas_call(
    _inproj_body,
    grid=(NP // RB,),
    in_specs=[pl.BlockSpec((RB, D), lambda i: (i, 0)),
              _node_spec((D, D)), _node_spec((1, D))],
    out_specs=pl.BlockSpec((RB, D), lambda i: (i, 0)),
    out_shape=jax.ShapeDtypeStruct((NP, D), jnp.float32),
)

_qkv = pl.pallas_call(
    _qkv_body,
    grid=(NP // RB,),
    in_specs=[pl.BlockSpec((RB, D), lambda i: (i, 0)),
              _node_spec((1, D)), _node_spec((1, D)),
              _node_spec((D, D)), _node_spec((1, D)),
              _node_spec((D, D)), _node_spec((1, D)),
              _node_spec((D, D)), _node_spec((1, D)),
              _node_spec((D, H))],
    out_specs=[pl.BlockSpec((RB, D), lambda i: (i, 0)),
               pl.BlockSpec((RB, D), lambda i: (i, 0)),
               pl.BlockSpec((RB, D), lambda i: (i, 0)),
               pl.BlockSpec((RB, H), lambda i: (i, 0)),
               pl.BlockSpec((RB, H), lambda i: (i, 0))],
    out_shape=[jax.ShapeDtypeStruct((NP, D), jnp.float32),
               jax.ShapeDtypeStruct((NP, D), jnp.float32),
               jax.ShapeDtypeStruct((NP, D), jnp.float32),
               jax.ShapeDtypeStruct((NP, H), jnp.float32),
               jax.ShapeDtypeStruct((NP, H), jnp.float32)],
)

_post = pl.pallas_call(
    _post_body,
    grid=(NP // RB,),
    in_specs=[pl.BlockSpec((RB, D), lambda i: (i, 0)),
              pl.BlockSpec((RB, D), lambda i: (i, 0)),
              pl.BlockSpec((RB, H), lambda i: (i, 0)),
              _node_spec((H, D)),
              _node_spec((D, D)), _node_spec((1, D)),
              _node_spec((1, D)), _node_spec((1, D)),
              _node_spec((D, D)), _node_spec((1, D)),
              _node_spec((D, D)), _node_spec((1, D))],
    out_specs=pl.BlockSpec((RB, D), lambda i: (i, 0)),
    out_shape=jax.ShapeDtypeStruct((NP, D), jnp.float32),
)

_mlp = pl.pallas_call(
    _mlp_body,
    grid=(EP // TB,),
    in_specs=[pl.BlockSpec((TB, D), lambda i: (i, 0)),
              pl.BlockSpec((TB, D), lambda i: (i, 0)),
              pl.BlockSpec((TB, 1), lambda i: (i, 0)),
              pl.BlockSpec((TB, 1), lambda i: (i, 0)),
              pl.BlockSpec((TB, 1), lambda i: (i, 0)),
              pl.BlockSpec((TB, 8), lambda i: (i, 0)),
              _node_spec((D, MP)), _node_spec((D, MP)),
              _node_spec((16, MP)), _node_spec((16, MP)),
              _node_spec((8, MP)), _node_spec((8, MP)), _node_spec((1, MP)),
              _node_spec((MP, MP)), _node_spec((1, MP)),
              _node_spec((MP, MP)), _node_spec((1, MP)),
              _node_spec((MP, 8)), _node_spec((1, 8))],
    out_specs=pl.BlockSpec((TB, 8), lambda i: (i, 0)),
    out_shape=jax.ShapeDtypeStruct((EP, 8), jnp.float32),
)


def _attn_sc_body(q2_hbm, k2_hbm, v2_hbm, src_hbm, dst_hbm, m_hbm,
                  z80_hbm, accden_out,
                  is0, id0, is1, id1, is2, id2, is3, id3,
                  qb0, kb0, vb0, wb0, qb1, kb1, vb1, wb1, mb,
                  sh, gsem0, gsem1, ssem0, ssem1):
    c = lax.axis_index("c")
    s = lax.axis_index("s")
    rows = NP // 16
    pltpu.sync_copy(z80_hbm.at[pl.ds(s * rows, rows)],
                    sh.at[pl.ds(s * rows, rows)])
    pltpu.sync_copy(m_hbm, mb)
    plsc.subcore_barrier()

    lanes = lax.iota(jnp.int32, 16)
    mvec = mb[...]

    def lgather(vec, idx):
        return lax.gather(
            vec, idx.reshape(16, 1),
            lax.GatherDimensionNumbers(offset_dims=(),
                                       collapsed_slice_dims=(0,),
                                       start_index_map=(0,)),
            (1,), mode=lax.GatherScatterMode.PROMISE_IN_BOUNDS)

    mh = [lgather(mvec, jnp.full((16,), 4 * c + h, jnp.int32))
          for h in range(H // 2)]
    perms = [lanes ^ (1 << b) for b in range(4)]
    inv = 0.25

    def allsum(p):
        for pm in perms:
            p = p + lgather(p, pm)
        return p

    idxb = [(is0, id0), (is1, id1), (is2, id2), (is3, id3)]
    datb = [(qb0, kb0, vb0, wb0, gsem0, ssem0),
            (qb1, kb1, vb1, wb1, gsem1, ssem1)]
    base0 = s * (EP // 16)

    def fire(ch, i4, p2):
        idxs_v, idxd_v = idxb[i4]
        qb, kb, vb, wb, gsem, ssem = datb[p2]
        base = base0 + ch * CA
        pltpu.sync_copy(src_hbm.at[pl.ds(base, CA)], idxs_v)
        pltpu.sync_copy(dst_hbm.at[pl.ds(base, CA)], idxd_v)
        pltpu.async_copy(q2_hbm.at[c].at[idxd_v], qb, gsem)
        pltpu.async_copy(k2_hbm.at[c].at[idxs_v], kb, gsem)
        pltpu.async_copy(v2_hbm.at[c].at[idxs_v], vb, gsem)

    def drain_scatter(i4, p2):
        idxs_v, idxd_v = idxb[i4]
        qb, kb, vb, wb, gsem, ssem = datb[p2]
        pltpu.make_async_copy(wb, sh.at[idxd_v], ssem).wait()

    def consume(ch, i4, p2, do_drain, do_fire):
        idxs_v, idxd_v = idxb[i4]
        qb, kb, vb, wb, gsem, ssem = datb[p2]
        pltpu.make_async_copy(q2_hbm.at[c].at[idxd_v], qb, gsem).wait()
        pltpu.make_async_copy(k2_hbm.at[c].at[idxs_v], kb, gsem).wait()
        pltpu.make_async_copy(v2_hbm.at[c].at[idxs_v], vb, gsem).wait()
        if do_drain:
            drain_scatter((i4 + 2) % 4, p2)

        def edge_body(e, _):
            ev = jnp.zeros((16,), jnp.float32)
            for h in range(H // 2):
                sl = pl.ds(h * DH, DH)
                p = qb[e, sl] * kb[e, sl]
                exv = jnp.exp(allsum(p) * inv - mh[h])
                wb[e, sl] = exv * vb[e, sl]
                ev = jnp.where(lanes == h, exv, ev)
            wb[e, pl.ds(64, 16)] = ev
            return 0

        lax.fori_loop(0, CA, edge_body, 0)
        pltpu.async_copy(wb, sh.at[idxd_v], sem=ssem, add=True)
        if do_fire:
            @pl.when(ch + 2 < NCH2)
            def _():
                fire(ch + 2, (i4 + 2) % 4, p2)

    fire(0, 0, 0)
    fire(1, 1, 1)
    consume(0, 0, 0, False, True)
    consume(1, 1, 1, False, True)

    def outer(ch4, carry):
        chb = ch4 * 4 + 2
        for j in range(4):
            consume(chb + j, (2 + j) % 4, j % 2, True, True)
        return carry

    lax.fori_loop(0, (NCH2 - 2) // 4, outer, 0)
    consume(NCH2 - 2, (NCH2 - 2) % 4, 0, True, False)
    consume(NCH2 - 1, (NCH2 - 1) % 4, 1, True, False)
    drain_scatter((NCH2 - 2) % 4, 0)
    drain_scatter((NCH2 - 1) % 4, 1)
    plsc.subcore_barrier()
    pltpu.sync_copy(sh.at[pl.ds(s * rows, rows)],
                    accden_out.at[c, pl.ds(s * rows, rows)])


@functools.cache
def _attn_kernel():
  idxset = [pltpu.VMEM((CA,), jnp.int32)] * 8
  datset = [pltpu.VMEM((CA, 64), jnp.float32),
            pltpu.VMEM((CA, 64), jnp.float32),
            pltpu.VMEM((CA, 64), jnp.float32),
            pltpu.VMEM((CA, 80), jnp.float32)] * 2
  return pl.kernel(
    _attn_sc_body,
    out_type=jax.ShapeDtypeStruct((2, NP, 80), jnp.float32),
    mesh=plsc.VectorSubcoreMesh(core_axis_name="c", subcore_axis_name="s"),
    compiler_params=pltpu.CompilerParams(use_tc_tiling_on_sc=False),
    scratch_types=idxset + datset + [
        pltpu.VMEM((16,), jnp.float32),
        pltpu.VMEM_SHARED((NP, 80), jnp.float32),
        pltpu.SemaphoreType.DMA,
        pltpu.SemaphoreType.DMA,
        pltpu.SemaphoreType.DMA,
        pltpu.SemaphoreType.DMA,
    ],
  )


def _gather_sc_body(h_hbm, src_hbm, dst_hbm, hs_out, hd_out,
                    is0, id0, is1, id1, is2, id2, is3, id3,
                    bs0, bd0, bs1, bd1, bs2, bd2, bs3, bd3,
                    g0, g1, g2, g3, w0, w1, w2, w3):
    c = lax.axis_index("c")
    s = lax.axis_index("s")
    wid = c * 16 + s
    base0 = wid * EW
    sets = [(is0, id0, bs0, bd0, g0, w0), (is1, id1, bs1, bd1, g1, w1),
            (is2, id2, bs2, bd2, g2, w2), (is3, id3, bs3, bd3, g3, w3)]

    def fire(ch, i4):
        idxs_v, idxd_v, bs, bd, gsem, wsem = sets[i4]
        base = base0 + ch * CA
        pltpu.sync_copy(src_hbm.at[pl.ds(base, CA)], idxs_v)
        pltpu.sync_copy(dst_hbm.at[pl.ds(base, CA)], idxd_v)
        pltpu.async_copy(h_hbm.at[idxs_v], bs, gsem)
        pltpu.async_copy(h_hbm.at[idxd_v], bd, gsem)

    def drain_write(ch, i4):
        idxs_v, idxd_v, bs, bd, gsem, wsem = sets[i4]
        bb = base0 + ch * CA
        pltpu.make_async_copy(bs, hs_out.at[pl.ds(bb, CA)], wsem).wait()
        pltpu.make_async_copy(bd, hd_out.at[pl.ds(bb, CA)], wsem).wait()

    def consume(ch, i4, do_drain, do_fire):
        idxs_v, idxd_v, bs, bd, gsem, wsem = sets[i4]
        base = base0 + ch * CA
        pltpu.make_async_copy(h_hbm.at[idxs_v], bs, gsem).wait()
        pltpu.make_async_copy(h_hbm.at[idxd_v], bd, gsem).wait()
        if do_drain:
            drain_write(ch - 2, (i4 + 2) % 4)
        pltpu.async_copy(bs, hs_out.at[pl.ds(base, CA)], wsem)
        pltpu.async_copy(bd, hd_out.at[pl.ds(base, CA)], wsem)
        if do_fire:
            @pl.when(ch + 2 < NCH)
            def _():
                fire(ch + 2, (i4 + 2) % 4)

    fire(0, 0)
    fire(1, 1)
    consume(0, 0, False, True)
    consume(1, 1, False, True)

    def outer(ch4, carry):
        chb = ch4 * 4 + 2
        for j in range(4):
            consume(chb + j, (2 + j) % 4, True, True)
        return carry

    lax.fori_loop(0, (NCH - 2) // 4, outer, 0)
    consume(NCH - 2, (NCH - 2) % 4, True, False)
    consume(NCH - 1, (NCH - 1) % 4, True, False)
    drain_write(NCH - 2, (NCH - 2) % 4)
    drain_write(NCH - 1, (NCH - 1) % 4)


@functools.cache
def _gather_kernel():
  return pl.kernel(
    _gather_sc_body,
    out_type=[jax.ShapeDtypeStruct((EP, D), jnp.float32),
              jax.ShapeDtypeStruct((EP, D), jnp.float32)],
    mesh=plsc.VectorSubcoreMesh(core_axis_name="c", subcore_axis_name="s"),
    scratch_types=(
        [pltpu.VMEM((CA,), jnp.int32)] * 8
        + [pltpu.VMEM((CA, D), jnp.float32)] * 8
        + [pltpu.SemaphoreType.DMA] * 8
    ),
  )


def kernel(x, numericals, Wh, bh, WQ, bQ, WK, bK, WV, bV, WO, bO,
           ln1_g, ln1_b, ln2_g, ln2_b, W1, b1, W2, b2, curr_emb, pay_emb,
           mlpW, mlpb, mlpWo, mlpbo, edge_index, payment_currency,
           receiving_currency, payment_format):
    f32 = jnp.float32
    xp = jnp.zeros((NP, D), f32).at[:N].set(x)
    src = edge_index[0]
    dst = edge_index[1]
    pad_e = EP - E
    srcp = jnp.concatenate([src, jnp.zeros((pad_e,), jnp.int32)])
    dstp = jnp.concatenate(
        [dst, N + (jnp.arange(pad_e, dtype=jnp.int32) % 16)])

    hsel = jnp.kron(jnp.eye(H, dtype=f32), jnp.ones((DH, 1), f32))
    r8 = jnp.kron(jnp.eye(H, dtype=f32), jnp.ones((1, DH), f32))
    z80 = jnp.zeros((NP, 80), f32)

    h = _inproj(xp, Wh, bh.reshape(1, D))

    for l in range(L):
        q, k, v, qn, kn = _qkv(h, ln1_g[l].reshape(1, D), ln1_b[l].reshape(1, D),
                               WQ[l], bQ[l].reshape(1, D),
                               WK[l], bK[l].reshape(1, D),
                               WV[l], bV[l].reshape(1, D), hsel)
        m8 = jnp.sqrt(jnp.max(qn, axis=0)) * jnp.sqrt(jnp.max(kn, axis=0)) * 0.25
        mvec = jnp.concatenate([m8, jnp.zeros((8,), f32)])
        q2 = jnp.stack([q[:, :64], q[:, 64:]])
        k2 = jnp.stack([k[:, :64], k[:, 64:]])
        v2 = jnp.stack([v[:, :64], v[:, 64:]])
        accden = _attn_kernel()(q2, k2, v2, srcp, dstp, mvec, z80)
        acc = jnp.concatenate([accden[0, :, :64], accden[1, :, :64]], axis=1)
        den = jnp.concatenate([accden[0, :, 64:68], accden[1, :, 64:68]], axis=1)
        h = _post(h, acc, den, r8,
                  WO[l], bO[l].reshape(1, D),
                  ln2_g[l].reshape(1, D), ln2_b[l].reshape(1, D),
                  W1[l], b1[l].reshape(1, D),
                  W2[l], b2[l].reshape(1, D))

    hs, hd = _gather_kernel()(h, srcp, dstp)

    w0 = mlpW[0]
    w0a = jnp.zeros((D, MP), f32).at[:, :285].set(w0[:D])
    w0b = jnp.zeros((D, MP), f32).at[:, :285].set(w0[D:2 * D])
    wpc = jnp.zeros((16, MP), f32).at[:15, :285].set(curr_emb @ w0[256:264])
    wrc = jnp.zeros((16, MP), f32).at[:15, :285].set(curr_emb @ w0[264:272])
    wpf = jnp.zeros((8, MP), f32).at[:7, :285].set(pay_emb @ w0[272:280])
    wnum = jnp.zeros((8, MP), f32).at[:5, :285].set(w0[280:285])
    b0 = jnp.zeros((1, MP), f32).at[0, :285].set(mlpb[0])
    w1m = jnp.zeros((MP, MP), f32).at[:285, :285].set(mlpW[1])
    b1m = jnp.zeros((1, MP), f32).at[0, :285].set(mlpb[1])
    w2m = jnp.zeros((MP, MP), f32).at[:285, :285].set(mlpW[2])
    b2m = jnp.zeros((1, MP), f32).at[0, :285].set(mlpb[2])
    wom = jnp.zeros((MP, 8), f32).at[:285, :2].set(mlpWo)
    bom = jnp.zeros((1, 8), f32).at[0, :2].set(mlpbo)

    pcp = jnp.concatenate([payment_currency,
                           jnp.zeros((pad_e,), jnp.int32)]).reshape(EP, 1)
    rcp = jnp.concatenate([receiving_currency,
                           jnp.zeros((pad_e,), jnp.int32)]).reshape(EP, 1)
    pfp = jnp.concatenate([payment_format,
                           jnp.zeros((pad_e,), jnp.int32)]).reshape(EP, 1)
    nump = jnp.zeros((EP, 8), f32).at[:E, :5].set(numericals)

    y = _mlp(hs, hd, pcp, rcp, pfp, nump,
             w0a, w0b, wpc, wrc, wpf, wnum, b0,
             w1m, b1m, w2m, b2m, wom, bom)
    return y[:E, :2]


# speedup vs baseline: 25.4054x; 1.8829x over previous
import functools

import jax
import jax.numpy as jnp
from jax import lax
from jax.experimental import pallas as pl
from jax.experimental.pallas import tpu as pltpu
from jax.experimental.pallas import tpu_sc as plsc

N = 10000
E = 160000
D = 128
H = 8
DH = 16
L = 4
NP = 10240
EP = 163840
NW = 32
EW = EP // NW
CA = 40
NCH = EW // CA
NCH2 = (EP // 16) // CA
RB = 1024
TB = 640
MP = 288


def _inproj_body(x_ref, w_ref, b_ref, o_ref):
    o_ref[...] = jnp.dot(x_ref[...], w_ref[...],
                         preferred_element_type=jnp.float32) + b_ref[...]


def _qkv_body(h_ref, g_ref, b_ref, wq_ref, bq_ref, wk_ref, bk_ref,
              wv_ref, bv_ref, hsel_ref, q_ref, k_ref, v_ref,
              qn_ref, kn_ref):
    h = h_ref[...]
    mu = jnp.mean(h, axis=1, keepdims=True)
    var = jnp.mean((h - mu) ** 2, axis=1, keepdims=True)
    hn = (h - mu) / jnp.sqrt(var + 1e-5) * g_ref[...] + b_ref[...]
    q = jnp.dot(hn, wq_ref[...], preferred_element_type=jnp.float32) + bq_ref[...]
    k = jnp.dot(hn, wk_ref[...], preferred_element_type=jnp.float32) + bk_ref[...]
    v = jnp.dot(hn, wv_ref[...], preferred_element_type=jnp.float32) + bv_ref[...]
    q_ref[...] = q
    k_ref[...] = k
    v_ref[...] = v
    hsel = hsel_ref[...]
    qn_ref[...] = jnp.dot(q * q, hsel, preferred_element_type=jnp.float32)
    kn_ref[...] = jnp.dot(k * k, hsel, preferred_element_type=jnp.float32)


def _post_body(h_ref, acc_ref, den_ref, r8_ref, wo_ref, bo_ref,
               g2_ref, b2_ref, w1_ref, b1_ref, w2_ref, b2m_ref, o_ref):
    acc = acc_ref[...]
    den = den_ref[...]
    denf = jnp.dot(den, r8_ref[...], preferred_element_type=jnp.float32)
    safe = jnp.where(denf > 0.0, denf, 1.0)
    hmha = acc / safe
    u = h_ref[...] + jnp.dot(hmha, wo_ref[...],
                             preferred_element_type=jnp.float32) + bo_ref[...]
    mu = jnp.mean(u, axis=1, keepdims=True)
    var = jnp.mean((u - mu) ** 2, axis=1, keepdims=True)
    n = (u - mu) / jnp.sqrt(var + 1e-5) * g2_ref[...] + b2_ref[...]
    y = jnp.maximum(jnp.dot(n, w1_ref[...],
                            preferred_element_type=jnp.float32) + b1_ref[...], 0.0)
    y = jnp.dot(y, w2_ref[...], preferred_element_type=jnp.float32) + b2m_ref[...]
    o_ref[...] = u + y


def _mlp_body(hs_ref, hd_ref, pc_ref, rc_ref, pf_ref, num_ref,
              w0a_ref, w0b_ref, wpc_ref, wrc_ref, wpf_ref, wnum_ref, b0_ref,
              w1_ref, b1_ref, w2_ref, b2_ref, wo_ref, bo_ref, o_ref):
    f32 = jnp.float32
    it16 = lax.broadcasted_iota(jnp.int32, (TB, 16), 1)
    it8 = lax.broadcasted_iota(jnp.int32, (TB, 8), 1)
    pcoh = (it16 == pc_ref[...]).astype(f32)
    rcoh = (it16 == rc_ref[...]).astype(f32)
    pfoh = (it8 == pf_ref[...]).astype(f32)
    y = (jnp.dot(hs_ref[...], w0a_ref[...], preferred_element_type=f32)
         + jnp.dot(hd_ref[...], w0b_ref[...], preferred_element_type=f32)
         + jnp.dot(pcoh, wpc_ref[...], preferred_element_type=f32)
         + jnp.dot(rcoh, wrc_ref[...], preferred_element_type=f32)
         + jnp.dot(pfoh, wpf_ref[...], preferred_element_type=f32)
         + jnp.dot(num_ref[...], wnum_ref[...], preferred_element_type=f32)
         + b0_ref[...])
    y = jnp.maximum(y, 0.0)
    y = jnp.maximum(jnp.dot(y, w1_ref[...], preferred_element_type=f32)
                    + b1_ref[...], 0.0)
    y = jnp.maximum(jnp.dot(y, w2_ref[...], preferred_element_type=f32)
                    + b2_ref[...], 0.0)
    o_ref[...] = jnp.dot(y, wo_ref[...], preferred_element_type=f32) + bo_ref[...]


def _node_spec(shape):
    return pl.BlockSpec(shape, lambda i: (0,) * len(shape))


_inproj = pl.pallas_call(
    _inproj_body,
    grid=(NP // RB,),
    in_specs=[pl.BlockSpec((RB, D), lambda i: (i, 0)),
              _node_spec((D, D)), _node_spec((1, D))],
    out_specs=pl.BlockSpec((RB, D), lambda i: (i, 0)),
    out_shape=jax.ShapeDtypeStruct((NP, D), jnp.float32),
)

_qkv = pl.pallas_call(
    _qkv_body,
    grid=(NP // RB,),
    in_specs=[pl.BlockSpec((RB, D), lambda i: (i, 0)),
              _node_spec((1, D)), _node_spec((1, D)),
              _node_spec((D, D)), _node_spec((1, D)),
              _node_spec((D, D)), _node_spec((1, D)),
              _node_spec((D, D)), _node_spec((1, D)),
              _node_spec((D, H))],
    out_specs=[pl.BlockSpec((RB, D), lambda i: (i, 0)),
               pl.BlockSpec((RB, D), lambda i: (i, 0)),
               pl.BlockSpec((RB, D), lambda i: (i, 0)),
               pl.BlockSpec((RB, H), lambda i: (i, 0)),
               pl.BlockSpec((RB, H), lambda i: (i, 0))],
    out_shape=[jax.ShapeDtypeStruct((NP, D), jnp.float32),
               jax.ShapeDtypeStruct((NP, D), jnp.float32),
               jax.ShapeDtypeStruct((NP, D), jnp.float32),
               jax.ShapeDtypeStruct((NP, H), jnp.float32),
               jax.ShapeDtypeStruct((NP, H), jnp.float32)],
)

_post = pl.pallas_call(
    _post_body,
    grid=(NP // RB,),
    in_specs=[pl.BlockSpec((RB, D), lambda i: (i, 0)),
              pl.BlockSpec((RB, D), lambda i: (i, 0)),
              pl.BlockSpec((RB, H), lambda i: (i, 0)),
              _node_spec((H, D)),
              _node_spec((D, D)), _node_spec((1, D)),
              _node_spec((1, D)), _node_spec((1, D)),
              _node_spec((D, D)), _node_spec((1, D)),
              _node_spec((D, D)), _node_spec((1, D))],
    out_specs=pl.BlockSpec((RB, D), lambda i: (i, 0)),
    out_shape=jax.ShapeDtypeStruct((NP, D), jnp.float32),
)

_mlp = pl.pallas_call(
    _mlp_body,
    grid=(EP // TB,),
    in_specs=[pl.BlockSpec((TB, D), lambda i: (i, 0)),
              pl.BlockSpec((TB, D), lambda i: (i, 0)),
              pl.BlockSpec((TB, 1), lambda i: (i, 0)),
              pl.BlockSpec((TB, 1), lambda i: (i, 0)),
              pl.BlockSpec((TB, 1), lambda i: (i, 0)),
              pl.BlockSpec((TB, 8), lambda i: (i, 0)),
              _node_spec((D, MP)), _node_spec((D, MP)),
              _node_spec((16, MP)), _node_spec((16, MP)),
              _node_spec((8, MP)), _node_spec((8, MP)), _node_spec((1, MP)),
              _node_spec((MP, MP)), _node_spec((1, MP)),
              _node_spec((MP, MP)), _node_spec((1, MP)),
              _node_spec((MP, 8)), _node_spec((1, 8))],
    out_specs=pl.BlockSpec((TB, 8), lambda i: (i, 0)),
    out_shape=jax.ShapeDtypeStruct((EP, 8), jnp.float32),
)


def _attn_sc_body(q2_hbm, k2_hbm, v2_hbm, src_hbm, dst_hbm, m_hbm,
                  z80_hbm, accden_out,
                  is0, id0, is1, id1, is2, id2, is3, id3,
                  qb0, kb0, vb0, wb0, qb1, kb1, vb1, wb1, mb,
                  sh, gsem0, gsem1, ssem0, ssem1):
    c = lax.axis_index("c")
    s = lax.axis_index("s")
    rows = NP // 16
    pltpu.sync_copy(z80_hbm.at[pl.ds(s * rows, rows)],
                    sh.at[pl.ds(s * rows, rows)])
    pltpu.sync_copy(m_hbm, mb)
    plsc.subcore_barrier()

    lanes = lax.iota(jnp.int32, 16)
    mvec = mb[...]

    def lgather(vec, idx):
        return lax.gather(
            vec, idx.reshape(16, 1),
            lax.GatherDimensionNumbers(offset_dims=(),
                                       collapsed_slice_dims=(0,),
                                       start_index_map=(0,)),
            (1,), mode=lax.GatherScatterMode.PROMISE_IN_BOUNDS)

    mh = [lgather(mvec, jnp.full((16,), 4 * c + h, jnp.int32))
          for h in range(H // 2)]
    perms = [lanes ^ (1 << b) for b in range(4)]
    inv = 0.25

    def allsum(p):
        for pm in perms:
            p = p + lgather(p, pm)
        return p

    idxb = [(is0, id0), (is1, id1), (is2, id2), (is3, id3)]
    datb = [(qb0, kb0, vb0, wb0, gsem0, ssem0),
            (qb1, kb1, vb1, wb1, gsem1, ssem1)]
    base0 = s * (EP // 16)

    def fire(ch, i4, p2):
        idxs_v, idxd_v = idxb[i4]
        qb, kb, vb, wb, gsem, ssem = datb[p2]
        base = base0 + ch * CA
        pltpu.sync_copy(src_hbm.at[pl.ds(base, CA)], idxs_v)
        pltpu.sync_copy(dst_hbm.at[pl.ds(base, CA)], idxd_v)
        pltpu.async_copy(q2_hbm.at[c].at[idxd_v], qb, gsem)
        pltpu.async_copy(k2_hbm.at[c].at[idxs_v], kb, gsem)
        pltpu.async_copy(v2_hbm.at[c].at[idxs_v], vb, gsem)

    def drain_scatter(i4, p2):
        idxs_v, idxd_v = idxb[i4]
        qb, kb, vb, wb, gsem, ssem = datb[p2]
        pltpu.make_async_copy(wb, sh.at[idxd_v], ssem).wait()

    def consume(ch, i4, p2, do_drain, do_fire):
        idxs_v, idxd_v = idxb[i4]
        qb, kb, vb, wb, gsem, ssem = datb[p2]
        pltpu.make_async_copy(q2_hbm.at[c].at[idxd_v], qb, gsem).wait()
        pltpu.make_async_copy(k2_hbm.at[c].at[idxs_v], kb, gsem).wait()
        pltpu.make_async_copy(v2_hbm.at[c].at[idxs_v], vb, gsem).wait()
        if do_drain:
            drain_scatter((i4 + 2) % 4, p2)

        @plsc.parallel_loop(0, CA, unroll=4)
        def _(e):
            ev = jnp.zeros((16,), jnp.float32)
            for h in range(H // 2):
                sl = pl.ds(h * DH, DH)
                p = qb[e, sl] * kb[e, sl]
                exv = jnp.exp(allsum(p) * inv - mh[h])
                wb[e, sl] = exv * vb[e, sl]
                ev = jnp.where(lanes == h, exv, ev)
            wb[e, pl.ds(64, 16)] = ev
        pltpu.async_copy(wb, sh.at[idxd_v], sem=ssem, add=True)
        if do_fire:
            @pl.when(ch + 2 < NCH2)
            def _():
                fire(ch + 2, (i4 + 2) % 4, p2)

    fire(0, 0, 0)
    fire(1, 1, 1)
    consume(0, 0, 0, False, True)
    consume(1, 1, 1, False, True)

    def outer(ch4, carry):
        chb = ch4 * 4 + 2
        for j in range(4):
            consume(chb + j, (2 + j) % 4, j % 2, True, True)
        return carry

    lax.fori_loop(0, (NCH2 - 2) // 4, outer, 0)
    consume(NCH2 - 2, (NCH2 - 2) % 4, 0, True, False)
    consume(NCH2 - 1, (NCH2 - 1) % 4, 1, True, False)
    drain_scatter((NCH2 - 2) % 4, 0)
    drain_scatter((NCH2 - 1) % 4, 1)
    plsc.subcore_barrier()
    pltpu.sync_copy(sh.at[pl.ds(s * rows, rows)],
                    accden_out.at[c, pl.ds(s * rows, rows)])


@functools.cache
def _attn_kernel():
  idxset = [pltpu.VMEM((CA,), jnp.int32)] * 8
  datset = [pltpu.VMEM((CA, 64), jnp.float32),
            pltpu.VMEM((CA, 64), jnp.float32),
            pltpu.VMEM((CA, 64), jnp.float32),
            pltpu.VMEM((CA, 80), jnp.float32)] * 2
  return pl.kernel(
    _attn_sc_body,
    out_type=jax.ShapeDtypeStruct((2, NP, 80), jnp.float32),
    mesh=plsc.VectorSubcoreMesh(core_axis_name="c", subcore_axis_name="s"),
    compiler_params=pltpu.CompilerParams(use_tc_tiling_on_sc=False),
    scratch_types=idxset + datset + [
        pltpu.VMEM((16,), jnp.float32),
        pltpu.VMEM_SHARED((NP, 80), jnp.float32),
        pltpu.SemaphoreType.DMA,
        pltpu.SemaphoreType.DMA,
        pltpu.SemaphoreType.DMA,
        pltpu.SemaphoreType.DMA,
    ],
  )


def _gather_sc_body(h_hbm, src_hbm, dst_hbm, hs_out, hd_out,
                    is0, id0, is1, id1, is2, id2, is3, id3,
                    bs0, bd0, bs1, bd1, bs2, bd2, bs3, bd3,
                    g0, g1, g2, g3, w0, w1, w2, w3):
    c = lax.axis_index("c")
    s = lax.axis_index("s")
    wid = c * 16 + s
    base0 = wid * EW
    sets = [(is0, id0, bs0, bd0, g0, w0), (is1, id1, bs1, bd1, g1, w1),
            (is2, id2, bs2, bd2, g2, w2), (is3, id3, bs3, bd3, g3, w3)]

    def fire(ch, i4):
        idxs_v, idxd_v, bs, bd, gsem, wsem = sets[i4]
        base = base0 + ch * CA
        pltpu.sync_copy(src_hbm.at[pl.ds(base, CA)], idxs_v)
        pltpu.sync_copy(dst_hbm.at[pl.ds(base, CA)], idxd_v)
        pltpu.async_copy(h_hbm.at[idxs_v], bs, gsem)
        pltpu.async_copy(h_hbm.at[idxd_v], bd, gsem)

    def drain_write(ch, i4):
        idxs_v, idxd_v, bs, bd, gsem, wsem = sets[i4]
        bb = base0 + ch * CA
        pltpu.make_async_copy(bs, hs_out.at[pl.ds(bb, CA)], wsem).wait()
        pltpu.make_async_copy(bd, hd_out.at[pl.ds(bb, CA)], wsem).wait()

    def consume(ch, i4, do_drain, do_fire):
        idxs_v, idxd_v, bs, bd, gsem, wsem = sets[i4]
        base = base0 + ch * CA
        pltpu.make_async_copy(h_hbm.at[idxs_v], bs, gsem).wait()
        pltpu.make_async_copy(h_hbm.at[idxd_v], bd, gsem).wait()
        if do_drain:
            drain_write(ch - 2, (i4 + 2) % 4)
        pltpu.async_copy(bs, hs_out.at[pl.ds(base, CA)], wsem)
        pltpu.async_copy(bd, hd_out.at[pl.ds(base, CA)], wsem)
        if do_fire:
            @pl.when(ch + 2 < NCH)
            def _():
                fire(ch + 2, (i4 + 2) % 4)

    fire(0, 0)
    fire(1, 1)
    consume(0, 0, False, True)
    consume(1, 1, False, True)

    def outer(ch4, carry):
        chb = ch4 * 4 + 2
        for j in range(4):
            consume(chb + j, (2 + j) % 4, True, True)
        return carry

    lax.fori_loop(0, (NCH - 2) // 4, outer, 0)
    consume(NCH - 2, (NCH - 2) % 4, True, False)
    consume(NCH - 1, (NCH - 1) % 4, True, False)
    drain_write(NCH - 2, (NCH - 2) % 4)
    drain_write(NCH - 1, (NCH - 1) % 4)


@functools.cache
def _gather_kernel():
  return pl.kernel(
    _gather_sc_body,
    out_type=[jax.ShapeDtypeStruct((EP, D), jnp.float32),
              jax.ShapeDtypeStruct((EP, D), jnp.float32)],
    mesh=plsc.VectorSubcoreMesh(core_axis_name="c", subcore_axis_name="s"),
    scratch_types=(
        [pltpu.VMEM((CA,), jnp.int32)] * 8
        + [pltpu.VMEM((CA, D), jnp.float32)] * 8
        + [pltpu.SemaphoreType.DMA] * 8
    ),
  )


def kernel(x, numericals, Wh, bh, WQ, bQ, WK, bK, WV, bV, WO, bO,
           ln1_g, ln1_b, ln2_g, ln2_b, W1, b1, W2, b2, curr_emb, pay_emb,
           mlpW, mlpb, mlpWo, mlpbo, edge_index, payment_currency,
           receiving_currency, payment_format):
    f32 = jnp.float32
    xp = jnp.zeros((NP, D), f32).at[:N].set(x)
    src = edge_index[0]
    dst = edge_index[1]
    pad_e = EP - E
    srcp = jnp.concatenate([src, jnp.zeros((pad_e,), jnp.int32)])
    dstp = jnp.concatenate(
        [dst, N + (jnp.arange(pad_e, dtype=jnp.int32) % 16)])

    hsel = jnp.kron(jnp.eye(H, dtype=f32), jnp.ones((DH, 1), f32))
    r8 = jnp.kron(jnp.eye(H, dtype=f32), jnp.ones((1, DH), f32))
    z80 = jnp.zeros((NP, 80), f32)

    h = _inproj(xp, Wh, bh.reshape(1, D))

    for l in range(L):
        q, k, v, qn, kn = _qkv(h, ln1_g[l].reshape(1, D), ln1_b[l].reshape(1, D),
                               WQ[l], bQ[l].reshape(1, D),
                               WK[l], bK[l].reshape(1, D),
                               WV[l], bV[l].reshape(1, D), hsel)
        m8 = jnp.sqrt(jnp.max(qn, axis=0)) * jnp.sqrt(jnp.max(kn, axis=0)) * 0.25
        mvec = jnp.concatenate([m8, jnp.zeros((8,), f32)])
        q2 = jnp.stack([q[:, :64], q[:, 64:]])
        k2 = jnp.stack([k[:, :64], k[:, 64:]])
        v2 = jnp.stack([v[:, :64], v[:, 64:]])
        accden = _attn_kernel()(q2, k2, v2, srcp, dstp, mvec, z80)
        acc = jnp.concatenate([accden[0, :, :64], accden[1, :, :64]], axis=1)
        den = jnp.concatenate([accden[0, :, 64:68], accden[1, :, 64:68]], axis=1)
        h = _post(h, acc, den, r8,
                  WO[l], bO[l].reshape(1, D),
                  ln2_g[l].reshape(1, D), ln2_b[l].reshape(1, D),
                  W1[l], b1[l].reshape(1, D),
                  W2[l], b2[l].reshape(1, D))

    hs, hd = _gather_kernel()(h, srcp, dstp)

    w0 = mlpW[0]
    w0a = jnp.zeros((D, MP), f32).at[:, :285].set(w0[:D])
    w0b = jnp.zeros((D, MP), f32).at[:, :285].set(w0[D:2 * D])
    wpc = jnp.zeros((16, MP), f32).at[:15, :285].set(curr_emb @ w0[256:264])
    wrc = jnp.zeros((16, MP), f32).at[:15, :285].set(curr_emb @ w0[264:272])
    wpf = jnp.zeros((8, MP), f32).at[:7, :285].set(pay_emb @ w0[272:280])
    wnum = jnp.zeros((8, MP), f32).at[:5, :285].set(w0[280:285])
    b0 = jnp.zeros((1, MP), f32).at[0, :285].set(mlpb[0])
    w1m = jnp.zeros((MP, MP), f32).at[:285, :285].set(mlpW[1])
    b1m = jnp.zeros((1, MP), f32).at[0, :285].set(mlpb[1])
    w2m = jnp.zeros((MP, MP), f32).at[:285, :285].set(mlpW[2])
    b2m = jnp.zeros((1, MP), f32).at[0, :285].set(mlpb[2])
    wom = jnp.zeros((MP, 8), f32).at[:285, :2].set(mlpWo)
    bom = jnp.zeros((1, 8), f32).at[0, :2].set(mlpbo)

    pcp = jnp.concatenate([payment_currency,
                           jnp.zeros((pad_e,), jnp.int32)]).reshape(EP, 1)
    rcp = jnp.concatenate([receiving_currency,
                           jnp.zeros((pad_e,), jnp.int32)]).reshape(EP, 1)
    pfp = jnp.concatenate([payment_format,
                           jnp.zeros((pad_e,), jnp.int32)]).reshape(EP, 1)
    nump = jnp.zeros((EP, 8), f32).at[:E, :5].set(numericals)

    y = _mlp(hs, hd, pcp, rcp, pfp, nump,
             w0a, w0b, wpc, wrc, wpf, wnum, b0,
             w1m, b1m, w2m, b2m, wom, bom)
    return y[:E, :2]
